# baseline (speedup 1.0000x reference)
import functools

import jax
import jax.numpy as jnp
from jax import lax
from jax.experimental import pallas as pl
from jax.experimental.pallas import tpu as pltpu
from jax.experimental.pallas import tpu_sc as plsc

N = 10000
E = 320000
D_IN = 128
D_H = 128
D_OUT = 64
DH2 = 64

NC = 2
NS = 16
B = 128
C1 = 160
C2 = 80
NBUF = 4
N_PAD = 10240
ROWS_PER_TILE = N_PAD // NS
BLK = 1000
GRID = N // BLK


def _make_segsum(D, C, with_deg):
  mesh = plsc.VectorSubcoreMesh(core_axis_name="c", subcore_axis_name="s")
  out_type = [jax.ShapeDtypeStruct((NC, N_PAD, D), jnp.float32)]
  if with_deg:
    out_type.append(jax.ShapeDtypeStruct((NS, N_PAD), jnp.float32))
  scratch = (
      [pltpu.VMEM((C, B), jnp.int32),
       pltpu.VMEM((C, B), jnp.int32)]
      + [pltpu.VMEM((B, D), jnp.float32)] * NBUF
      + [pltpu.VMEM_SHARED((N_PAD, D), jnp.float32)]
      + [pltpu.SemaphoreType.DMA] * (2 * NBUF)
  )
  if with_deg:
    scratch.append(pltpu.VMEM((N_PAD,), jnp.float32))

  @functools.partial(
      pl.kernel, mesh=mesh, out_type=out_type, scratch_types=scratch,
      compiler_params=pltpu.CompilerParams(needs_layout_passes=False,
                                           use_tc_tiling_on_sc=False))
  def seg(*refs):
    if with_deg:
      (table_h, srcb_h, dstb_h, zrows_h, zdeg_h, out_h, deg_h,
       src_v, dst_v, *rest) = refs
      deg_v = rest[-1]
      rest = rest[:-1]
    else:
      (table_h, srcb_h, dstb_h, zrows_h, out_h, src_v, dst_v, *rest) = refs
    rows = rest[:NBUF]
    acc_sh = rest[NBUF]
    semg = rest[NBUF + 1:2 * NBUF + 1]
    sems = rest[2 * NBUF + 1:3 * NBUF + 1]
    cid = lax.axis_index("c")
    sid = lax.axis_index("s")
    r0 = sid * ROWS_PER_TILE

    pltpu.sync_copy(zrows_h, acc_sh.at[pl.ds(r0, ROWS_PER_TILE)])
    pltpu.sync_copy(srcb_h.at[cid, sid], src_v)
    pltpu.sync_copy(dstb_h.at[cid, sid], dst_v)
    if with_deg:
      pltpu.sync_copy(zdeg_h, deg_v)
    plsc.subcore_barrier()

    ones = jnp.ones((16,), jnp.float32)

    for b in range(NBUF):
      pltpu.async_copy(table_h.at[src_v.at[b]], rows[b], semg[b])

    def chunkn(jj, carry):
      base = jj * NBUF
      scat = []
      for b in range(NBUF):
        j = base + b
        pltpu.make_async_copy(table_h.at[src_v.at[j]], rows[b],
                              semg[b]).wait()
        scat.append(pltpu.async_copy(rows[b], acc_sh.at[dst_v.at[j]],
                                     sems[b], add=True))
      if with_deg:
        @pl.when(cid == 0)
        def _():
          for b in range(NBUF):
            for k in range(B // 16):
              idx16 = dst_v[base + b, k * 16:(k + 1) * 16]
              plsc.addupdate_scatter(deg_v, [idx16], ones)
      for b in range(NBUF):
        scat[b].wait()

        @pl.when(base + b + NBUF < C)
        def _(b=b):
          pltpu.async_copy(table_h.at[src_v.at[base + b + NBUF]], rows[b],
                           semg[b])

      return carry

    lax.fori_loop(0, C // NBUF, chunkn, 0)
    plsc.subcore_barrier()

    pltpu.sync_copy(acc_sh.at[pl.ds(r0, ROWS_PER_TILE)],
                    out_h.at[cid].at[pl.ds(r0, ROWS_PER_TILE)])
    if with_deg:
      @pl.when(cid == 0)
      def _():
        pltpu.sync_copy(deg_v, deg_h.at[sid])

  return seg


_segsum1 = _make_segsum(DH2, C1, with_deg=True)
_segsum2 = _make_segsum(D_OUT // 2, C1, with_deg=False)


def _dense1_body(x_ref, agg_ref, degp_ref, ws1_ref, wn1_ref, b1_ref,
                 ws2_ref, wn2_ref, b2_ref, t_ref, s_ref):
  deg = jnp.sum(degp_ref[...], axis=1)
  dinv = 1.0 / jnp.maximum(deg, 1.0)
  hn = jnp.concatenate([agg_ref[0], agg_ref[1]], axis=1) * dinv[:, None]
  h1 = (jnp.dot(x_ref[...], ws1_ref[...], preferred_element_type=jnp.float32)
        + jnp.dot(hn, wn1_ref[...], preferred_element_type=jnp.float32)
        + b1_ref[...])
  h1 = jnp.maximum(h1, 0.0)
  t_ref[...] = jnp.dot(h1, wn2_ref[...], preferred_element_type=jnp.float32)
  s_ref[...] = (jnp.dot(h1, ws2_ref[...], preferred_element_type=jnp.float32)
                + b2_ref[...])


def _dense1(x, agg1, degp, ws1, wn1, b1, ws2, wn2, b2):
  return pl.pallas_call(
      _dense1_body,
      grid=(GRID,),
      in_specs=[
          pl.BlockSpec((BLK, D_IN), lambda i: (i, 0)),
          pl.BlockSpec((NC, BLK, DH2), lambda i: (0, i, 0)),
          pl.BlockSpec((BLK, NS), lambda i: (i, 0)),
          pl.BlockSpec((D_IN, D_H), lambda i: (0, 0)),
          pl.BlockSpec((D_IN, D_H), lambda i: (0, 0)),
          pl.BlockSpec((1, D_H), lambda i: (0, 0)),
          pl.BlockSpec((D_H, D_OUT), lambda i: (0, 0)),
          pl.BlockSpec((D_H, D_OUT), lambda i: (0, 0)),
          pl.BlockSpec((1, D_OUT), lambda i: (0, 0)),
      ],
      out_specs=[
          pl.BlockSpec((BLK, D_OUT), lambda i: (i, 0)),
          pl.BlockSpec((BLK, D_OUT), lambda i: (i, 0)),
      ],
      out_shape=[
          jax.ShapeDtypeStruct((N, D_OUT), jnp.float32),
          jax.ShapeDtypeStruct((N, D_OUT), jnp.float32),
      ],
  )(x, agg1, degp, ws1, wn1, b1, ws2, wn2, b2)


def _dense2_body(s_ref, agg_ref, degp_ref, o_ref):
  deg = jnp.sum(degp_ref[...], axis=1)
  dinv = 1.0 / jnp.maximum(deg, 1.0)
  agg = jnp.concatenate([agg_ref[0], agg_ref[1]], axis=1)
  o_ref[...] = s_ref[...] + agg * dinv[:, None]


def _dense2(souts, agg2, degp):
  return pl.pallas_call(
      _dense2_body,
      grid=(GRID,),
      in_specs=[
          pl.BlockSpec((BLK, D_OUT), lambda i: (i, 0)),
          pl.BlockSpec((NC, BLK, D_OUT // 2), lambda i: (0, i, 0)),
          pl.BlockSpec((BLK, NS), lambda i: (i, 0)),
      ],
      out_specs=pl.BlockSpec((BLK, D_OUT), lambda i: (i, 0)),
      out_shape=jax.ShapeDtypeStruct((N, D_OUT), jnp.float32),
  )(souts, agg2, degp)


def kernel(x, edge_index, W_self1, W_neigh1, b1, W_self2, W_neigh2, b2):
  src = edge_index[0].astype(jnp.int32)
  dst = edge_index[1].astype(jnp.int32)

  pad1 = NS * C1 * B - E
  trash1 = N + (jnp.arange(pad1, dtype=jnp.int32) % (N_PAD - N))

  src1 = jnp.concatenate([src, jnp.zeros((pad1,), jnp.int32)]).reshape(
      NS, C1, B)
  dst1 = jnp.concatenate([dst, trash1]).reshape(NS, C1, B)
  src1 = jnp.stack([src1, src1 + N])
  dst1 = jnp.stack([dst1, dst1])
  x2 = jnp.concatenate([x[:, :DH2], x[:, DH2:]], axis=0)

  zrows1 = jnp.zeros((ROWS_PER_TILE, DH2), jnp.float32)
  zrows2 = jnp.zeros((ROWS_PER_TILE, D_OUT // 2), jnp.float32)
  zdeg = jnp.zeros((N_PAD,), jnp.float32)

  agg1, degp = _segsum1(x2, src1, dst1, zrows1, zdeg)
  degp = jnp.transpose(degp)
  t, souts = _dense1(x, agg1, degp, W_self1, W_neigh1,
                     b1.reshape(1, -1), W_self2, W_neigh2, b2.reshape(1, -1))
  t2 = jnp.concatenate([t[:, :D_OUT // 2], t[:, D_OUT // 2:]], axis=0)
  (agg2,) = _segsum2(t2, src1, dst1, zrows2)
  return _dense2(souts, agg2, degp)


# speedup vs baseline: 1.2308x; 1.2308x over previous
import functools

import jax
import jax.numpy as jnp
from jax import lax
from jax.experimental import pallas as pl
from jax.experimental.pallas import tpu as pltpu
from jax.experimental.pallas import tpu_sc as plsc

N = 10000
E = 320000
D_IN = 128
D_H = 128
D_OUT = 64
DH2 = 64

NC = 2
NS = 16
B = 128
C1 = 158
C2 = 80
NBUF = 2
N_PAD = 10240
ROWS_PER_TILE = N_PAD // NS
BLK = 1000
GRID = N // BLK


def _make_segsum(D, C, with_deg):
  mesh = plsc.VectorSubcoreMesh(core_axis_name="c", subcore_axis_name="s")
  out_type = [jax.ShapeDtypeStruct((NC, N_PAD, D), jnp.float32)]
  if with_deg:
    out_type.append(jax.ShapeDtypeStruct((NS, N_PAD), jnp.float32))
  scratch = (
      [pltpu.VMEM((C, B), jnp.int32),
       pltpu.VMEM((C, B), jnp.int32)]
      + [pltpu.VMEM((B, D), jnp.float32)] * NBUF
      + [pltpu.VMEM_SHARED((N_PAD, D), jnp.float32)]
      + [pltpu.SemaphoreType.DMA] * (2 * NBUF)
  )
  if with_deg:
    scratch.append(pltpu.VMEM((N_PAD,), jnp.float32))

  @functools.partial(
      pl.kernel, mesh=mesh, out_type=out_type, scratch_types=scratch,
      compiler_params=pltpu.CompilerParams(needs_layout_passes=False,
                                           use_tc_tiling_on_sc=False))
  def seg(*refs):
    if with_deg:
      (table_h, srcb_h, dstb_h, zrows_h, zdeg_h, out_h, deg_h,
       src_v, dst_v, *rest) = refs
      deg_v = rest[-1]
      rest = rest[:-1]
    else:
      (table_h, srcb_h, dstb_h, zrows_h, out_h, src_v, dst_v, *rest) = refs
    rows = rest[:NBUF]
    acc_sh = rest[NBUF]
    semg = rest[NBUF + 1:2 * NBUF + 1]
    sems = rest[2 * NBUF + 1:3 * NBUF + 1]
    cid = lax.axis_index("c")
    sid = lax.axis_index("s")
    r0 = sid * ROWS_PER_TILE

    pltpu.sync_copy(zrows_h, acc_sh.at[pl.ds(r0, ROWS_PER_TILE)])
    pltpu.sync_copy(srcb_h.at[cid, sid], src_v)
    pltpu.sync_copy(dstb_h.at[sid], dst_v)
    if with_deg:
      pltpu.sync_copy(zdeg_h, deg_v)
    plsc.subcore_barrier()

    ones = jnp.ones((16,), jnp.float32)

    for b in range(NBUF):
      pltpu.async_copy(table_h.at[src_v.at[b]], rows[b], semg[b])

    def chunkn(jj, carry):
      base = jj * NBUF
      scat = []
      for b in range(NBUF):
        j = base + b
        pltpu.make_async_copy(table_h.at[src_v.at[j]], rows[b],
                              semg[b]).wait()
        scat.append(pltpu.async_copy(rows[b], acc_sh.at[dst_v.at[j]],
                                     sems[b], add=True))
      if with_deg:
        @pl.when(cid == 0)
        def _():
          for b in range(NBUF):
            for k in range(B // 16):
              idx16 = dst_v[base + b, k * 16:(k + 1) * 16]
              plsc.addupdate_scatter(deg_v, [idx16], ones)
      for b in range(NBUF):
        scat[b].wait()

        @pl.when(base + b + NBUF < C)
        def _(b=b):
          pltpu.async_copy(table_h.at[src_v.at[base + b + NBUF]], rows[b],
                           semg[b])

      return carry

    lax.fori_loop(0, C // NBUF, chunkn, 0)
    plsc.subcore_barrier()

    pltpu.sync_copy(acc_sh.at[pl.ds(r0, ROWS_PER_TILE)],
                    out_h.at[cid].at[pl.ds(r0, ROWS_PER_TILE)])
    if with_deg:
      @pl.when(cid == 0)
      def _():
        pltpu.sync_copy(deg_v, deg_h.at[sid])

  return seg


_segsum1 = _make_segsum(DH2, C1, with_deg=True)
_segsum2 = _make_segsum(D_OUT // 2, C1, with_deg=False)


def _dense1_body(x_ref, agg_ref, degp_ref, ws1_ref, wn1_ref, b1_ref,
                 ws2_ref, wn2_ref, b2_ref, t_ref, s_ref):
  deg = jnp.sum(degp_ref[...], axis=1)
  dinv = 1.0 / jnp.maximum(deg, 1.0)
  hn = jnp.concatenate([agg_ref[0], agg_ref[1]], axis=1) * dinv[:, None]
  h1 = (jnp.dot(x_ref[...], ws1_ref[...], preferred_element_type=jnp.float32)
        + jnp.dot(hn, wn1_ref[...], preferred_element_type=jnp.float32)
        + b1_ref[...])
  h1 = jnp.maximum(h1, 0.0)
  tt = jnp.dot(h1, wn2_ref[...], preferred_element_type=jnp.float32)
  t_ref[0] = tt[:, :D_OUT // 2]
  t_ref[1] = tt[:, D_OUT // 2:]
  s_ref[...] = (jnp.dot(h1, ws2_ref[...], preferred_element_type=jnp.float32)
                + b2_ref[...])


def _dense1(x, agg1, degp, ws1, wn1, b1, ws2, wn2, b2):
  return pl.pallas_call(
      _dense1_body,
      grid=(GRID,),
      in_specs=[
          pl.BlockSpec((BLK, D_IN), lambda i: (i, 0)),
          pl.BlockSpec((NC, BLK, DH2), lambda i: (0, i, 0)),
          pl.BlockSpec((BLK, NS), lambda i: (i, 0)),
          pl.BlockSpec((D_IN, D_H), lambda i: (0, 0)),
          pl.BlockSpec((D_IN, D_H), lambda i: (0, 0)),
          pl.BlockSpec((1, D_H), lambda i: (0, 0)),
          pl.BlockSpec((D_H, D_OUT), lambda i: (0, 0)),
          pl.BlockSpec((D_H, D_OUT), lambda i: (0, 0)),
          pl.BlockSpec((1, D_OUT), lambda i: (0, 0)),
      ],
      out_specs=[
          pl.BlockSpec((NC, BLK, D_OUT // 2), lambda i: (0, i, 0)),
          pl.BlockSpec((BLK, D_OUT), lambda i: (i, 0)),
      ],
      out_shape=[
          jax.ShapeDtypeStruct((NC, N, D_OUT // 2), jnp.float32),
          jax.ShapeDtypeStruct((N, D_OUT), jnp.float32),
      ],
  )(x, agg1, degp, ws1, wn1, b1, ws2, wn2, b2)


def _dense2_body(s_ref, agg_ref, degp_ref, o_ref):
  deg = jnp.sum(degp_ref[...], axis=1)
  dinv = 1.0 / jnp.maximum(deg, 1.0)
  agg = jnp.concatenate([agg_ref[0], agg_ref[1]], axis=1)
  o_ref[...] = s_ref[...] + agg * dinv[:, None]


def _dense2(souts, agg2, degp):
  return pl.pallas_call(
      _dense2_body,
      grid=(GRID,),
      in_specs=[
          pl.BlockSpec((BLK, D_OUT), lambda i: (i, 0)),
          pl.BlockSpec((NC, BLK, D_OUT // 2), lambda i: (0, i, 0)),
          pl.BlockSpec((BLK, NS), lambda i: (i, 0)),
      ],
      out_specs=pl.BlockSpec((BLK, D_OUT), lambda i: (i, 0)),
      out_shape=jax.ShapeDtypeStruct((N, D_OUT), jnp.float32),
  )(souts, agg2, degp)


def kernel(x, edge_index, W_self1, W_neigh1, b1, W_self2, W_neigh2, b2):
  src = edge_index[0].astype(jnp.int32)
  dst = edge_index[1].astype(jnp.int32)

  pad1 = NS * C1 * B - E
  trash1 = N + (jnp.arange(pad1, dtype=jnp.int32) % (N_PAD - N))

  src1 = jnp.concatenate([src, jnp.zeros((pad1,), jnp.int32)]).reshape(
      NS, C1, B)
  dst1 = jnp.concatenate([dst, trash1]).reshape(NS, C1, B)
  src1 = jnp.stack([src1, src1 + N])
  x2 = jnp.concatenate([x[:, :DH2], x[:, DH2:]], axis=0)

  zrows1 = jnp.zeros((ROWS_PER_TILE, DH2), jnp.float32)
  zrows2 = jnp.zeros((ROWS_PER_TILE, D_OUT // 2), jnp.float32)
  zdeg = jnp.zeros((N_PAD,), jnp.float32)

  agg1, degp = _segsum1(x2, src1, dst1, zrows1, zdeg)
  degp = jnp.transpose(degp)
  t, souts = _dense1(x, agg1, degp, W_self1, W_neigh1,
                     b1.reshape(1, -1), W_self2, W_neigh2, b2.reshape(1, -1))
  (agg2,) = _segsum2(t.reshape(NC * N, D_OUT // 2), src1, dst1, zrows2)
  return _dense2(souts, agg2, degp)
